# Initial kernel scaffold; baseline (speedup 1.0000x reference)
import jax
import jax.numpy as jnp
from jax.experimental import pallas as pl


def kernel(x, edge_index, values, bias):
    raise NotImplementedError("write your pallas kernel here")


# speedup vs baseline: 36.0462x; 36.0462x over previous
import functools

import jax
import jax.numpy as jnp
from jax import lax
from jax.experimental import pallas as pl
from jax.experimental.pallas import tpu as pltpu
from jax.experimental.pallas import tpu_sc as plsc

NC = 2
NS = 16
L = 16
CH = 128


def _sc_scatter(xT, src2, dst2, val2, zinit, *, cpt, m_pad):
    rpt = m_pad // NS

    mesh = plsc.VectorSubcoreMesh(core_axis_name="c", subcore_axis_name="s")

    @functools.partial(
        pl.kernel,
        out_type=jax.ShapeDtypeStruct((NC, m_pad, L), jnp.float32),
        mesh=mesh,
        scratch_types=[
            pltpu.VMEM_SHARED((m_pad, L), jnp.float32),
            pltpu.VMEM((CH,), jnp.int32),
            pltpu.VMEM((1, CH), jnp.int32),
            pltpu.VMEM((CH,), jnp.float32),
            pltpu.VMEM((CH, L), jnp.float32),
            pltpu.SemaphoreType.DMA,
        ],
        compiler_params=pltpu.CompilerParams(use_tc_tiling_on_sc=False),
    )
    def body(xT_h, src_h, dst_h, val_h, zin_h, part_h,
             acc, sbuf, dbuf, vbuf, xrow, gsem):
        c = lax.axis_index("c")
        s = lax.axis_index("s")
        wid = s * NC + c

        pltpu.sync_copy(zin_h.at[pl.ds(s * rpt, rpt)], acc.at[pl.ds(s * rpt, rpt)])
        plsc.subcore_barrier()

        base = wid * cpt

        def chunk(j, _):
            row = base + j
            pltpu.sync_copy(src_h.at[row], sbuf)
            pltpu.sync_copy(dst_h.at[pl.ds(row, 1)], dbuf)
            pltpu.sync_copy(val_h.at[row], vbuf)
            pltpu.async_copy(xT_h.at[sbuf], xrow, gsem).wait()

            def mul(g, _):
                gb = g * L
                vv = vbuf[pl.ds(gb, L)]
                for k in range(L):
                    xrow[gb + k, :] = vv[k] * xrow[gb + k, :]
                return 0

            lax.fori_loop(0, CH // L, mul, 0)
            pltpu.sync_copy(xrow, acc.at[dbuf.at[0]], add=True)
            return 0

        lax.fori_loop(0, cpt, chunk, 0)
        plsc.subcore_barrier()

        pltpu.sync_copy(acc.at[pl.ds(s * rpt, rpt)],
                        part_h.at[c, pl.ds(s * rpt, rpt)])

    return body(xT, src2, dst2, val2, zinit)


def _tc_combine(p0, p1, b16):
    r = p0.shape[0]
    br = r // 2
    assert r % br == 0 and br % 8 == 0

    def body(a_ref, b_ref, c_ref, o_ref):
        o_ref[...] = a_ref[...] + b_ref[...] + c_ref[...]

    return pl.pallas_call(
        body,
        out_shape=jax.ShapeDtypeStruct((r, 128), jnp.float32),
        grid=(r // br,),
        in_specs=[pl.BlockSpec((br, 128), lambda i: (i, 0))] * 3,
        out_specs=pl.BlockSpec((br, 128), lambda i: (i, 0)),
    )(p0, p1, b16)


def kernel(x, edge_index, values, bias):
    B, N, _ = x.shape
    M = bias.shape[0]
    E = values.shape[0]

    cpt = -(-E // (NC * NS * CH))
    e_pad = NC * NS * CH * cpt
    m_pad = -(-M // (NS * 8)) * (NS * 8)

    xT = x.reshape(B, N).T

    src = edge_index[0]
    dst = edge_index[1]
    pad = e_pad - E
    fill = jnp.arange(pad, dtype=jnp.int32)
    src_p = jnp.concatenate([src, fill % N])
    dst_p = jnp.concatenate([dst, fill % M])
    val_p = jnp.concatenate([values, jnp.zeros((pad,), jnp.float32)])
    src2 = src_p.reshape(-1, CH)
    dst2 = dst_p.reshape(-1, CH)
    val2 = val_p.reshape(-1, CH)
    zinit = jnp.zeros((m_pad, L), jnp.float32)

    part = _sc_scatter(xT, src2, dst2, val2, zinit, cpt=cpt, m_pad=m_pad)

    b16 = jnp.pad(jnp.broadcast_to(bias.reshape(M, 1), (M, L)),
                  ((0, m_pad - M), (0, 0)))
    out16 = _tc_combine(part[0].reshape(-1, 128), part[1].reshape(-1, 128),
                        b16.reshape(-1, 128))
    out16 = out16.reshape(m_pad, L)[:M]
    return out16.T.reshape(B, M, 1)


# speedup vs baseline: 74.8234x; 2.0758x over previous
import functools

import jax
import jax.numpy as jnp
from jax import lax
from jax.experimental import pallas as pl
from jax.experimental.pallas import tpu as pltpu
from jax.experimental.pallas import tpu_sc as plsc

NC = 2
NS = 16
L = 16
CH = 128
SB = 50


def _sc_scatter(xT, src2, dst2, val2, zinit, *, cpt, m_pad):
    rpt = m_pad // NS
    nblk = cpt // SB
    assert cpt % SB == 0 and SB % 2 == 0

    mesh = plsc.VectorSubcoreMesh(core_axis_name="c", subcore_axis_name="s")

    @functools.partial(
        pl.kernel,
        out_type=jax.ShapeDtypeStruct((NC, m_pad, L), jnp.float32),
        mesh=mesh,
        scratch_types=[
            pltpu.VMEM_SHARED((m_pad, L), jnp.float32),
            pltpu.VMEM((2, SB, CH), jnp.int32),
            pltpu.VMEM((2, SB, CH), jnp.int32),
            pltpu.VMEM((2, SB, CH), jnp.float32),
            pltpu.VMEM((2, CH, L), jnp.float32),
            pltpu.SemaphoreType.DMA,
            pltpu.SemaphoreType.DMA,
            pltpu.SemaphoreType.DMA,
        ],
        compiler_params=pltpu.CompilerParams(use_tc_tiling_on_sc=False),
    )
    def body(xT_h, src_h, dst_h, val_h, zin_h, part_h,
             acc, sgs, sgd, sgv, xr, stsem, gsem, ssem):
        c = lax.axis_index("c")
        s = lax.axis_index("s")
        wid = s * NC + c

        pltpu.sync_copy(zin_h.at[pl.ds(s * rpt, rpt)], acc.at[pl.ds(s * rpt, rpt)])
        plsc.subcore_barrier()

        base = wid * cpt

        def stage(blk, op):
            slot = lax.rem(blk, 2)
            row0 = base + blk * SB
            op(src_h.at[pl.ds(row0, SB)], sgs.at[slot], stsem)
            op(dst_h.at[pl.ds(row0, SB)], sgd.at[slot], stsem)
            op(val_h.at[pl.ds(row0, SB)], sgv.at[slot], stsem)

        stage_issue = lambda blk: stage(blk, pltpu.async_copy)
        stage_wait = lambda blk: stage(
            blk, lambda a, b, m: pltpu.make_async_copy(a, b, m).wait())

        stage_issue(0)

        def blk_body(blk, _):
            slot = lax.rem(blk, 2)
            stage_wait(blk)

            @pl.when(blk + 1 < nblk)
            def _():
                stage_issue(blk + 1)

            def g_copy(j):
                return xT_h.at[sgs.at[slot, j]], xr.at[lax.rem(j, 2)], gsem

            def s_copy(j):
                return xr.at[lax.rem(j, 2)], acc.at[sgd.at[slot, j]], ssem

            pltpu.async_copy(*g_copy(0))

            def chunk(j, _):
                pltpu.make_async_copy(*g_copy(j)).wait()

                @pl.when(j >= 1)
                def _():
                    pltpu.make_async_copy(*s_copy(j - 1)).wait()

                @pl.when(j + 1 < SB)
                def _():
                    pltpu.async_copy(*g_copy(j + 1))

                p = lax.rem(j, 2)

                def mul(g, _):
                    gb = g * L
                    vv = sgv[slot, j, pl.ds(gb, L)]
                    for k in range(L):
                        xr[p, gb + k, :] = vv[k] * xr[p, gb + k, :]
                    return 0

                lax.fori_loop(0, CH // L, mul, 0)
                pltpu.async_copy(*s_copy(j), add=True)
                return 0

            lax.fori_loop(0, SB, chunk, 0)
            pltpu.make_async_copy(*s_copy(SB - 1)).wait()
            return 0

        lax.fori_loop(0, nblk, blk_body, 0)
        plsc.subcore_barrier()

        pltpu.sync_copy(acc.at[pl.ds(s * rpt, rpt)],
                        part_h.at[c, pl.ds(s * rpt, rpt)])

    return body(xT, src2, dst2, val2, zinit)


def _tc_combine(p0, p1, b16):
    r = p0.shape[0]
    br = r // 2
    assert r % br == 0 and br % 8 == 0

    def body(a_ref, b_ref, c_ref, o_ref):
        o_ref[...] = a_ref[...] + b_ref[...] + c_ref[...]

    return pl.pallas_call(
        body,
        out_shape=jax.ShapeDtypeStruct((r, 128), jnp.float32),
        grid=(r // br,),
        in_specs=[pl.BlockSpec((br, 128), lambda i: (i, 0))] * 3,
        out_specs=pl.BlockSpec((br, 128), lambda i: (i, 0)),
    )(p0, p1, b16)


def kernel(x, edge_index, values, bias):
    B, N, _ = x.shape
    M = bias.shape[0]
    E = values.shape[0]

    cpt = -(-E // (NC * NS * CH))
    cpt = -(-cpt // SB) * SB
    e_pad = NC * NS * CH * cpt
    m_pad = -(-M // (NS * 8)) * (NS * 8)

    xT = x.reshape(B, N).T

    src = edge_index[0]
    dst = edge_index[1]
    pad = e_pad - E
    fill = jnp.arange(pad, dtype=jnp.int32)
    src_p = jnp.concatenate([src, fill % N])
    dst_p = jnp.concatenate([dst, fill % M])
    val_p = jnp.concatenate([values, jnp.zeros((pad,), jnp.float32)])
    src2 = src_p.reshape(-1, CH)
    dst2 = dst_p.reshape(-1, CH)
    val2 = val_p.reshape(-1, CH)
    zinit = jnp.zeros((m_pad, L), jnp.float32)

    part = _sc_scatter(xT, src2, dst2, val2, zinit, cpt=cpt, m_pad=m_pad)

    b16 = jnp.pad(jnp.broadcast_to(bias.reshape(M, 1), (M, L)),
                  ((0, m_pad - M), (0, 0)))
    out16 = _tc_combine(part[0].reshape(-1, 128), part[1].reshape(-1, 128),
                        b16.reshape(-1, 128))
    out16 = out16.reshape(m_pad, L)[:M]
    return out16.T.reshape(B, M, 1)
